# baseline (speedup 1.0000x reference)
import jax
import jax.numpy as jnp
from jax.experimental import pallas as pl
from jax.experimental.pallas import tpu as pltpu

E = 64
K = 8
DI1 = 64
DI2 = 32
DO1 = 64
DO2 = 32
DIN = DI1 * DI2
DOUT = DO1 * DO2

BLK = 64


def _topk_weights(logits):
    cur = logits
    top1 = jnp.max(cur, axis=-1, keepdims=True)
    wacc = jnp.zeros_like(logits)
    denom = jnp.zeros_like(top1)
    iota = jax.lax.broadcasted_iota(jnp.int32, logits.shape, 1)
    for _ in range(K):
        m = jnp.max(cur, axis=-1, keepdims=True)
        sel = cur == m
        midx = jnp.min(jnp.where(sel, iota, E), axis=-1, keepdims=True)
        first = iota == midx
        ev = jnp.exp(m - top1)
        wacc = wacc + jnp.where(first, ev, 0.0)
        denom = denom + ev
        cur = jnp.where(first, -jnp.inf, cur)
    return wacc / denom


def _moe_kernel(x_ref, x2_ref, wrt_ref, sb_ref, sa_ref, rep_ref, tile_ref,
                sc_ref, bias_ref, out_ref):
    m = x_ref.shape[0]
    xb = x_ref[...]

    logits = jnp.dot(xb, wrt_ref[...], preferred_element_type=jnp.float32)
    w = _topk_weights(logits)

    v = jnp.dot(x2_ref[...].astype(jnp.bfloat16), sb_ref[...],
                preferred_element_type=jnp.float32)

    wrow = jnp.dot(w, tile_ref[...], preferred_element_type=jnp.float32)
    wexp = jnp.dot(rep_ref[...], wrow, preferred_element_type=jnp.float32)
    vw = (v * wexp).astype(jnp.bfloat16).reshape(m, DI1, DO2, E)

    vt = vw.transpose(0, 2, 1, 3).reshape(m * DO2, DI1 * E)
    out = jnp.dot(vt, sa_ref[...], preferred_element_type=jnp.float32)

    res = out.reshape(m, DO2, DO1).transpose(0, 2, 1)
    out_ref[...] = res * sc_ref[0, 0] + bias_ref[...]


@jax.jit
def _run(xf, x2h, wrt, sb, sa, rep, tile, scale2, bias3):
    n = xf.shape[0]
    grid = (n // BLK,)
    return pl.pallas_call(
        _moe_kernel,
        grid=grid,
        in_specs=[
            pl.BlockSpec((BLK, DIN), lambda i: (i, 0)),
            pl.BlockSpec((BLK * DI1, DI2), lambda i: (i, 0)),
            pl.BlockSpec((DIN, E), lambda i: (0, 0)),
            pl.BlockSpec((DI2, DO2 * E), lambda i: (0, 0)),
            pl.BlockSpec((DI1 * E, DO1), lambda i: (0, 0)),
            pl.BlockSpec((BLK * DI1, BLK), lambda i: (0, 0)),
            pl.BlockSpec((E, DO2 * E), lambda i: (0, 0)),
            pl.BlockSpec((1, 1), lambda i: (0, 0)),
            pl.BlockSpec((1, DO1, DO2), lambda i: (0, 0, 0)),
        ],
        out_specs=pl.BlockSpec((BLK, DO1, DO2), lambda i: (i, 0, 0)),
        out_shape=jax.ShapeDtypeStruct((n, DO1, DO2), jnp.float32),
        compiler_params=pltpu.CompilerParams(
            vmem_limit_bytes=100 * 1024 * 1024),
    )(xf, x2h, wrt, sb, sa, rep, tile, scale2, bias3)


def kernel(x, Wr, A, B, scale, bias):
    orig_shape = x.shape
    xf = x.reshape(-1, DIN)
    wrt = Wr.T
    sb = B.transpose(2, 1, 0).reshape(DI2, DO2 * E).astype(jnp.bfloat16)
    sa = A.transpose(2, 0, 1).reshape(DI1 * E, DO1).astype(jnp.bfloat16)
    rep = (jnp.arange(BLK * DI1)[:, None] // DI1
           == jnp.arange(BLK)[None, :]).astype(jnp.float32)
    tile = (jnp.arange(E)[:, None]
            == jnp.arange(DO2 * E)[None, :] % E).astype(jnp.float32)
    out = _run(xf, x.reshape(-1, DI2), wrt, sb, sa, rep, tile,
               scale.reshape(1, 1), bias.reshape(1, DO1, DO2))
    out = out.reshape(*orig_shape[:-1], DOUT)
    aux_loss = jnp.asarray(0.0, dtype=x.dtype)
    return (out, aux_loss)


# speedup vs baseline: 1.2456x; 1.2456x over previous
import jax
import jax.numpy as jnp
from jax.experimental import pallas as pl
from jax.experimental.pallas import tpu as pltpu

E = 64
K = 8
DI1 = 64
DI2 = 32
DO1 = 64
DO2 = 32
DIN = DI1 * DI2
DOUT = DO1 * DO2

BLK = 32


def _topk_weights(logits):
    cur = logits
    top1 = jnp.max(cur, axis=-1, keepdims=True)
    wacc = jnp.zeros_like(logits)
    denom = jnp.zeros_like(top1)
    iota = jax.lax.broadcasted_iota(jnp.int32, logits.shape, 1)
    for _ in range(K):
        m = jnp.max(cur, axis=-1, keepdims=True)
        sel = cur == m
        midx = jnp.min(jnp.where(sel, iota, E), axis=-1, keepdims=True)
        first = iota == midx
        ev = jnp.exp(m - top1)
        wacc = wacc + jnp.where(first, ev, 0.0)
        denom = denom + ev
        cur = jnp.where(first, -jnp.inf, cur)
    return wacc / denom


def _moe_kernel(x_ref, x2_ref, wrt_ref, sb_ref, sa_ref, rep_ref, tile_ref,
                sc_ref, bias_ref, out_ref):
    m = x_ref.shape[0]
    xb = x_ref[...]

    logits = jnp.dot(xb, wrt_ref[...], preferred_element_type=jnp.float32)
    w = _topk_weights(logits)

    v = jnp.dot(x2_ref[...].astype(jnp.bfloat16), sb_ref[...],
                preferred_element_type=jnp.float32)

    wrow = jnp.dot(w, tile_ref[...], preferred_element_type=jnp.float32)
    wexp = jnp.dot(rep_ref[...], wrow, preferred_element_type=jnp.float32)
    vw = (v * wexp).astype(jnp.bfloat16).reshape(m, DI1, DO2, E)

    vt = vw.transpose(0, 2, 1, 3).reshape(m * DO2, DI1 * E)
    out = jnp.dot(vt, sa_ref[...], preferred_element_type=jnp.float32)

    res = out.reshape(m, DO2, DO1).transpose(0, 2, 1)
    out_ref[...] = res * sc_ref[0, 0] + bias_ref[...]


@jax.jit
def _run(xf, x2h, wrt, sb, sa, rep, tile, scale2, bias3):
    n = xf.shape[0]
    grid = (n // BLK,)
    return pl.pallas_call(
        _moe_kernel,
        grid=grid,
        in_specs=[
            pl.BlockSpec((BLK, DIN), lambda i: (i, 0)),
            pl.BlockSpec((BLK * DI1, DI2), lambda i: (i, 0)),
            pl.BlockSpec((DIN, E), lambda i: (0, 0)),
            pl.BlockSpec((DI2, DO2 * E), lambda i: (0, 0)),
            pl.BlockSpec((DI1 * E, DO1), lambda i: (0, 0)),
            pl.BlockSpec((BLK * DI1, BLK), lambda i: (0, 0)),
            pl.BlockSpec((E, DO2 * E), lambda i: (0, 0)),
            pl.BlockSpec((1, 1), lambda i: (0, 0)),
            pl.BlockSpec((1, DO1, DO2), lambda i: (0, 0, 0)),
        ],
        out_specs=pl.BlockSpec((BLK, DO1, DO2), lambda i: (i, 0, 0)),
        out_shape=jax.ShapeDtypeStruct((n, DO1, DO2), jnp.float32),
        compiler_params=pltpu.CompilerParams(
            vmem_limit_bytes=100 * 1024 * 1024),
    )(xf, x2h, wrt, sb, sa, rep, tile, scale2, bias3)


def kernel(x, Wr, A, B, scale, bias):
    orig_shape = x.shape
    xf = x.reshape(-1, DIN)
    wrt = Wr.T
    sb = B.transpose(2, 1, 0).reshape(DI2, DO2 * E).astype(jnp.bfloat16)
    sa = A.transpose(2, 0, 1).reshape(DI1 * E, DO1).astype(jnp.bfloat16)
    rep = (jnp.arange(BLK * DI1)[:, None] // DI1
           == jnp.arange(BLK)[None, :]).astype(jnp.float32)
    tile = (jnp.arange(E)[:, None]
            == jnp.arange(DO2 * E)[None, :] % E).astype(jnp.float32)
    out = _run(xf, x.reshape(-1, DI2), wrt, sb, sa, rep, tile,
               scale.reshape(1, 1), bias.reshape(1, DO1, DO2))
    out = out.reshape(*orig_shape[:-1], DOUT)
    aux_loss = jnp.asarray(0.0, dtype=x.dtype)
    return (out, aux_loss)


# speedup vs baseline: 1.2727x; 1.0218x over previous
import jax
import jax.numpy as jnp
from jax.experimental import pallas as pl
from jax.experimental.pallas import tpu as pltpu

E = 64
K = 8
DI1 = 64
DI2 = 32
DO1 = 64
DO2 = 32
DIN = DI1 * DI2
DOUT = DO1 * DO2

BLK = 32


def _topk_weights(logits):
    cur = logits
    top1 = jnp.max(cur, axis=-1, keepdims=True)
    wacc = jnp.zeros_like(logits)
    denom = jnp.zeros_like(top1)
    iota = jax.lax.broadcasted_iota(jnp.int32, logits.shape, 1)
    for _ in range(K):
        m = jnp.max(cur, axis=-1, keepdims=True)
        sel = cur == m
        midx = jnp.min(jnp.where(sel, iota, E), axis=-1, keepdims=True)
        first = iota == midx
        ev = jnp.exp(m - top1)
        wacc = wacc + jnp.where(first, ev, 0.0)
        denom = denom + ev
        cur = jnp.where(first, -jnp.inf, cur)
    return wacc / denom


def _moe_kernel(x_ref, x2_ref, wrt_ref, sb_ref, sa_ref, rep_ref, tile_ref,
                sc_ref, bias_ref, out_ref):
    m = x_ref.shape[0]
    xb = x_ref[...]

    logits = jnp.dot(xb, wrt_ref[...], preferred_element_type=jnp.float32)
    w = _topk_weights(logits)

    v = jnp.dot(x2_ref[...].astype(jnp.bfloat16), sb_ref[...],
                preferred_element_type=jnp.float32)

    wrow = jnp.dot(w, tile_ref[...], preferred_element_type=jnp.float32)
    vw = ((v.reshape(m, DI1, DO2 * E) * wrow.reshape(m, 1, DO2 * E))
          .astype(jnp.bfloat16).reshape(m, DI1, DO2, E))

    vt = vw.transpose(0, 2, 1, 3).reshape(m * DO2, DI1 * E)
    out = jnp.dot(vt, sa_ref[...], preferred_element_type=jnp.float32)

    res = out.reshape(m, DO2, DO1).transpose(0, 2, 1)
    out_ref[...] = res * sc_ref[0, 0] + bias_ref[...]


@jax.jit
def _run(xf, x2h, wrt, sb, sa, rep, tile, scale2, bias3):
    n = xf.shape[0]
    grid = (n // BLK,)
    return pl.pallas_call(
        _moe_kernel,
        grid=grid,
        in_specs=[
            pl.BlockSpec((BLK, DIN), lambda i: (i, 0)),
            pl.BlockSpec((BLK * DI1, DI2), lambda i: (i, 0)),
            pl.BlockSpec((DIN, E), lambda i: (0, 0)),
            pl.BlockSpec((DI2, DO2 * E), lambda i: (0, 0)),
            pl.BlockSpec((DI1 * E, DO1), lambda i: (0, 0)),
            pl.BlockSpec((BLK * DI1, BLK), lambda i: (0, 0)),
            pl.BlockSpec((E, DO2 * E), lambda i: (0, 0)),
            pl.BlockSpec((1, 1), lambda i: (0, 0)),
            pl.BlockSpec((1, DO1, DO2), lambda i: (0, 0, 0)),
        ],
        out_specs=pl.BlockSpec((BLK, DO1, DO2), lambda i: (i, 0, 0)),
        out_shape=jax.ShapeDtypeStruct((n, DO1, DO2), jnp.float32),
        compiler_params=pltpu.CompilerParams(
            vmem_limit_bytes=100 * 1024 * 1024),
    )(xf, x2h, wrt, sb, sa, rep, tile, scale2, bias3)


def kernel(x, Wr, A, B, scale, bias):
    orig_shape = x.shape
    xf = x.reshape(-1, DIN)
    wrt = Wr.T
    sb = B.transpose(2, 1, 0).reshape(DI2, DO2 * E).astype(jnp.bfloat16)
    sa = A.transpose(2, 0, 1).reshape(DI1 * E, DO1).astype(jnp.bfloat16)
    rep = (jnp.arange(BLK * DI1)[:, None] // DI1
           == jnp.arange(BLK)[None, :]).astype(jnp.float32)
    tile = (jnp.arange(E)[:, None]
            == jnp.arange(DO2 * E)[None, :] % E).astype(jnp.float32)
    out = _run(xf, x.reshape(-1, DI2), wrt, sb, sa, rep, tile,
               scale.reshape(1, 1), bias.reshape(1, DO1, DO2))
    out = out.reshape(*orig_shape[:-1], DOUT)
    aux_loss = jnp.asarray(0.0, dtype=x.dtype)
    return (out, aux_loss)


# speedup vs baseline: 1.2751x; 1.0018x over previous
import jax
import jax.numpy as jnp
from jax.experimental import pallas as pl
from jax.experimental.pallas import tpu as pltpu

E = 64
K = 8
DI1 = 64
DI2 = 32
DO1 = 64
DO2 = 32
DIN = DI1 * DI2
DOUT = DO1 * DO2

BLK = 32


def _topk_weights(logits):
    cur = logits
    top1 = jnp.max(cur, axis=-1, keepdims=True)
    wacc = jnp.zeros_like(logits)
    denom = jnp.zeros_like(top1)
    iota = jax.lax.broadcasted_iota(jnp.int32, logits.shape, 1)
    for _ in range(K):
        m = jnp.max(cur, axis=-1, keepdims=True)
        sel = cur == m
        midx = jnp.min(jnp.where(sel, iota, E), axis=-1, keepdims=True)
        first = iota == midx
        ev = jnp.exp(m - top1)
        wacc = wacc + jnp.where(first, ev, 0.0)
        denom = denom + ev
        cur = jnp.where(first, -jnp.inf, cur)
    return wacc / denom


def _moe_kernel(x_ref, x2_ref, wrt_ref, sb_ref, sa_ref, tile_ref,
                sc_ref, bias_ref, out_ref):
    m = x_ref.shape[0]
    xb = x_ref[...]

    logits = jnp.dot(xb, wrt_ref[...], preferred_element_type=jnp.float32)
    w = _topk_weights(logits)

    v = jnp.dot(x2_ref[...].astype(jnp.bfloat16), sb_ref[...],
                preferred_element_type=jnp.float32)

    wrow = jnp.dot(w, tile_ref[...], preferred_element_type=jnp.float32)
    vw = ((v.reshape(m, DI1, DO2 * E) * wrow.reshape(m, 1, DO2 * E))
          .astype(jnp.bfloat16).reshape(m, DI1, DO2, E))

    vt = vw.transpose(0, 2, 1, 3).reshape(m * DO2, DI1 * E)
    out = jnp.dot(vt, sa_ref[...], preferred_element_type=jnp.float32)

    res = out.reshape(m, DO2, DO1).transpose(0, 2, 1)
    out_ref[...] = res * sc_ref[0, 0] + bias_ref[...]


@jax.jit
def _run(xf, x2h, wrt, sb, sa, tile, scale2, bias3):
    n = xf.shape[0]
    grid = (n // BLK,)
    return pl.pallas_call(
        _moe_kernel,
        grid=grid,
        in_specs=[
            pl.BlockSpec((BLK, DIN), lambda i: (i, 0)),
            pl.BlockSpec((BLK * DI1, DI2), lambda i: (i, 0)),
            pl.BlockSpec((DIN, E), lambda i: (0, 0)),
            pl.BlockSpec((DI2, DO2 * E), lambda i: (0, 0)),
            pl.BlockSpec((DI1 * E, DO1), lambda i: (0, 0)),
            pl.BlockSpec((E, DO2 * E), lambda i: (0, 0)),
            pl.BlockSpec((1, 1), lambda i: (0, 0)),
            pl.BlockSpec((1, DO1, DO2), lambda i: (0, 0, 0)),
        ],
        out_specs=pl.BlockSpec((BLK, DO1, DO2), lambda i: (i, 0, 0)),
        out_shape=jax.ShapeDtypeStruct((n, DO1, DO2), jnp.float32),
        compiler_params=pltpu.CompilerParams(
            vmem_limit_bytes=100 * 1024 * 1024),
    )(xf, x2h, wrt, sb, sa, tile, scale2, bias3)


def kernel(x, Wr, A, B, scale, bias):
    orig_shape = x.shape
    xf = x.reshape(-1, DIN)
    wrt = Wr.T
    sb = B.transpose(2, 1, 0).reshape(DI2, DO2 * E).astype(jnp.bfloat16)
    sa = A.transpose(2, 0, 1).reshape(DI1 * E, DO1).astype(jnp.bfloat16)
    tile = (jnp.arange(E)[:, None]
            == jnp.arange(DO2 * E)[None, :] % E).astype(jnp.float32)
    out = _run(xf, x.reshape(-1, DI2), wrt, sb, sa, tile,
               scale.reshape(1, 1), bias.reshape(1, DO1, DO2))
    out = out.reshape(*orig_shape[:-1], DOUT)
    aux_loss = jnp.asarray(0.0, dtype=x.dtype)
    return (out, aux_loss)
